# baseline (speedup 1.0000x reference)
import functools
import jax
import jax.numpy as jnp
from jax import lax
from jax.experimental import pallas as pl
from jax.experimental.pallas import tpu as pltpu
from jax.experimental.pallas import tpu_sc as plsc

NC, NS, L = 2, 16, 16
NW = NC * NS
NGRP = 16
NHALF = 2
NB = 4
PF = 2


def _sc_sample(T2, scal, wq0a, wq1a, yoa, wy0a, wy1a, B, H, W, C):
    P = H * W
    NI = H // NGRP
    JW = W // NHALF
    WW = JW + 2
    RWORDS = WW * C
    OWORDS = JW * C
    JPAD = wq0a.shape[-1]
    IPAD = yoa.shape[-1]

    mesh = plsc.VectorSubcoreMesh(core_axis_name="c", subcore_axis_name="s")

    @functools.partial(
        pl.kernel,
        mesh=mesh,
        out_type=jax.ShapeDtypeStruct((B * P * C,), jnp.float32),
        compiler_params=pltpu.CompilerParams(
            needs_layout_passes=False, use_tc_tiling_on_sc=False
        ),
        scratch_types=[
            pltpu.VMEM((16,), jnp.int32),
            pltpu.VMEM((JW * L,), jnp.float32),
            pltpu.VMEM((JW * L,), jnp.float32),
            pltpu.VMEM((IPAD,), jnp.int32),
            pltpu.VMEM((IPAD,), jnp.float32),
            pltpu.VMEM((IPAD,), jnp.float32),
            pltpu.VMEM((NB * RWORDS,), jnp.float32),
            pltpu.VMEM((2 * OWORDS,), jnp.float32),
            pltpu.SemaphoreType.DMA,
            pltpu.SemaphoreType.DMA,
        ],
    )
    def k(T2_hbm, scal_hbm, wq0_hbm, wq1_hbm, yo_hbm, wy0_hbm, wy1_hbm,
          out_hbm, scal_v, wq0ev, wq1ev, yov, wy0v, wy1v, RB, OB, rsem, osem):
        wid = lax.axis_index("s") * NC + lax.axis_index("c")
        g = wid // NHALF
        h = wid % NHALF
        i0 = g * NI
        j0 = h * JW

        def fire_row(b, sy, tx, r):
            src = T2_hbm.at[b * H + sy + r, pl.ds(tx * C, RWORDS)]
            pltpu.async_copy(src, RB.at[pl.ds((r % NB) * RWORDS, RWORDS)],
                             rsem)

        def wait_row(b):
            pltpu.make_async_copy(
                T2_hbm.at[b * H, pl.ds(0, RWORDS)],
                RB.at[pl.ds(0, RWORDS)], rsem
            ).wait()

        def fire_out(b, i_abs, p):
            dst = out_hbm.at[pl.ds((b * P + i_abs * W + j0) * C, OWORDS)]
            pltpu.async_copy(OB.at[pl.ds(p * OWORDS, OWORDS)], dst, osem)

        def drain_out():
            pltpu.make_async_copy(
                OB.at[pl.ds(0, OWORDS)], out_hbm.at[pl.ds(0, OWORDS)], osem
            ).wait()

        def batch(b, carry0):
            pltpu.sync_copy(scal_hbm.at[b, wid], scal_v)
            pltpu.sync_copy(wq0_hbm.at[b, h], wq0ev)
            pltpu.sync_copy(wq1_hbm.at[b, h], wq1ev)
            pltpu.sync_copy(yo_hbm.at[b, g], yov)
            pltpu.sync_copy(wy0_hbm.at[b, g], wy0v)
            pltpu.sync_copy(wy1_hbm.at[b, g], wy1v)
            sv = scal_v[pl.ds(0, L)]
            tx = sv[0]
            sy = sv[1]
            arel = sv[2]
            cl = sv[3]
            ch = sv[4]
            jlo8 = sv[5]
            ng = sv[6]
            jend = sv[7]
            lastv = yov[pl.ds(NI - 1, L)]
            cap = lastv[0] + 2

            def row(i_loc, carry):
                fired, waited = carry
                yv0 = yov[pl.ds(i_loc, L)]
                o0 = yv0[0]
                need = o0 + 2
                want = jnp.minimum(need + PF, cap)

                def fcond(s):
                    return s < want

                def fbody(s):
                    fire_row(b, sy, tx, s)
                    return s + 1

                fired = lax.while_loop(fcond, fbody, fired)

                def wcond(s):
                    return s < need

                def wbody(s):
                    wait_row(b)
                    return s + 1

                waited = lax.while_loop(wcond, wbody, waited)

                r0 = (o0 % NB) * RWORDS
                r1 = ((o0 + 1) % NB) * RWORDS
                wv0 = wy0v[pl.ds(i_loc, L)]
                wv1 = wy1v[pl.ds(i_loc, L)]
                gy0 = jnp.full((L,), wv0[0], dtype=jnp.float32)
                gy1 = jnp.full((L,), wv1[0], dtype=jnp.float32)
                p = i_loc % 2

                t_glob = b * NI + i_loc

                @pl.when(t_glob >= 2)
                def _():
                    drain_out()

                po = p * OWORDS

                def jone(jl, c2):
                    m = jnp.minimum(jnp.maximum(jl + arel, cl), ch) * C
                    t0 = wq0ev[pl.ds(jl * L, L)]
                    t1 = wq1ev[pl.ds(jl * L, L)]
                    wA = gy0 * t0
                    wB = gy1 * t0
                    wC = gy0 * t1
                    wD = gy1 * t1
                    ob = po + jl * C
                    for cc in range(C // L):
                        va = RB[pl.ds(r0 + m + cc * L, L)]
                        vb = RB[pl.ds(r1 + m + cc * L, L)]
                        vc = RB[pl.ds(r0 + m + C + cc * L, L)]
                        vd = RB[pl.ds(r1 + m + C + cc * L, L)]
                        acc = (wA * va + wB * vb) + (wC * vc + wD * vd)
                        OB[pl.ds(ob + cc * L, L)] = acc
                    return c2

                GRP = 16

                def jgroup(t, c2):
                    js = jlo8 + t * GRP
                    mb = (js + arel) * C
                    NCC = C // L
                    cur0 = [RB[pl.ds(r0 + mb + cc * L, L)]
                            for cc in range(NCC)]
                    cur1 = [RB[pl.ds(r1 + mb + cc * L, L)]
                            for cc in range(NCC)]
                    for kk in range(GRP):
                        nb = mb + (kk + 1) * C
                        nxt0 = [RB[pl.ds(r0 + nb + cc * L, L)]
                                for cc in range(NCC)]
                        nxt1 = [RB[pl.ds(r1 + nb + cc * L, L)]
                                for cc in range(NCC)]
                        jl = js + kk
                        t0 = wq0ev[pl.ds(jl * L, L)]
                        t1 = wq1ev[pl.ds(jl * L, L)]
                        wA = gy0 * t0
                        wB = gy1 * t0
                        wC = gy0 * t1
                        wD = gy1 * t1
                        ob = po + jl * C
                        for cc in range(NCC):
                            acc = (wA * cur0[cc] + wB * cur1[cc]) + \
                                (wC * nxt0[cc] + wD * nxt1[cc])
                            OB[pl.ds(ob + cc * L, L)] = acc
                        cur0 = nxt0
                        cur1 = nxt1
                    return c2

                lax.fori_loop(0, jlo8, jone, 0)
                lax.fori_loop(0, ng, jgroup, 0)
                lax.fori_loop(jend, JW, jone, 0)
                fire_out(b, i0 + i_loc, p)
                return (fired, waited)

            lax.fori_loop(0, NI, row, (jnp.int32(0), jnp.int32(0)))
            return carry0

        lax.fori_loop(0, B, batch, 0)
        drain_out()
        drain_out()

    return k(T2, scal, wq0a, wq1a, yoa, wy0a, wy1a)


def kernel(U, theta, out_size):
    B, C, H, W = U.shape
    oh, ow = H, W
    P = H * W
    NI = H // NGRP
    JW = W // NHALF
    WW = JW + 2
    NR = NI + 2
    zero = (jnp.asarray(out_size) - oh).astype(U.dtype)
    ox = jnp.linspace(-1.0, 1.0, ow)
    oy = jnp.linspace(-1.0, 1.0, oh)
    x = (theta[:, 0, 0][:, None] + ox[None, :]) + zero
    y = (theta[:, 1, 0][:, None] + oy[None, :]) + zero
    x = (x + 1.0) * (float(W) - 1.0) / 2.0
    y = (y + 1.0) * (float(H) - 1.0) / 2.0
    x0 = jnp.clip(jnp.floor(x).astype(jnp.int32), 0, W - 2)
    x1 = jnp.clip(jnp.ceil(x).astype(jnp.int32), 0, W - 1)
    y0 = jnp.clip(jnp.floor(y).astype(jnp.int32), 0, H - 2)
    y1 = jnp.clip(jnp.ceil(y).astype(jnp.int32), 0, H - 1)
    wx0 = x1.astype(x.dtype) - x
    wx1 = x - x0.astype(x.dtype)
    wy0 = y1.astype(y.dtype) - y
    wy1 = y - y0.astype(y.dtype)

    qoffx = jnp.floor(theta[:, 0, 0] * (float(W) - 1.0) / 2.0).astype(
        jnp.int32)
    qoffy = jnp.floor(theta[:, 1, 0] * (float(H) - 1.0) / 2.0).astype(
        jnp.int32)
    jj = jnp.arange(W, dtype=jnp.int32)
    ii = jnp.arange(H, dtype=jnp.int32)
    qx = jnp.clip(jj[None, :] + qoffx[:, None], 0, W - 2)
    qy = jnp.clip(ii[None, :] + qoffy[:, None], 0, H - 2)
    fz = jnp.float32(0.0)
    wq0 = jnp.where(x0 == qx, wx0, fz) + jnp.where(x1 == qx, wx1, fz)
    wq1 = jnp.where(x0 == qx + 1, wx0, fz) + jnp.where(x1 == qx + 1, wx1, fz)
    wyq0 = jnp.where(y0 == qy, wy0, fz) + jnp.where(y1 == qy, wy1, fz)
    wyq1 = jnp.where(y0 == qy + 1, wy0, fz) + jnp.where(y1 == qy + 1, wy1, fz)

    tx = jnp.minimum(qx[:, ::JW], W - WW)
    sy = jnp.minimum(qy[:, ::NI], H - NR)
    gg = jnp.arange(NW, dtype=jnp.int32) // NHALF
    hh = jnp.arange(NW, dtype=jnp.int32) % NHALF
    arel = hh[None, :] * JW + qoffx[:, None] - tx[:, hh]
    j0h = (hh * JW)[None, :]
    jlo_raw = jnp.clip(-qoffx[:, None] - j0h, 0, JW)
    jhi_raw = jnp.clip((W - 1) - qoffx[:, None] - j0h, 0, JW)
    jlo8 = jnp.minimum((jlo_raw + 15) // 16 * 16, JW)
    ngrp8 = jnp.maximum(jhi_raw - jlo8, 0) // 16
    jend = jlo8 + 16 * ngrp8
    scal = jnp.zeros((B, NW, 16), jnp.int32)
    scal = scal.at[:, :, 0].set(tx[:, hh])
    scal = scal.at[:, :, 1].set(sy[:, gg])
    scal = scal.at[:, :, 2].set(arel)
    scal = scal.at[:, :, 3].set(-tx[:, hh])
    scal = scal.at[:, :, 4].set((W - 2) - tx[:, hh])
    scal = scal.at[:, :, 5].set(jlo8)
    scal = scal.at[:, :, 6].set(ngrp8)
    scal = scal.at[:, :, 7].set(jend)

    wq0a = jnp.broadcast_to(
        wq0.reshape(B, NHALF, JW, 1), (B, NHALF, JW, 16)
    ).reshape(B, NHALF, JW * 16)
    wq1a = jnp.broadcast_to(
        wq1.reshape(B, NHALF, JW, 1), (B, NHALF, JW, 16)
    ).reshape(B, NHALF, JW * 16)

    IPAD = NI + 24
    yo = qy.reshape(B, NGRP, NI) - sy[:, :, None]
    yoa = jnp.zeros((B, NGRP, IPAD), jnp.int32).at[:, :, :NI].set(yo)
    wy0a = jnp.zeros((B, NGRP, IPAD), jnp.float32).at[:, :, :NI].set(
        wyq0.reshape(B, NGRP, NI))
    wy1a = jnp.zeros((B, NGRP, IPAD), jnp.float32).at[:, :, :NI].set(
        wyq1.reshape(B, NGRP, NI))

    T2 = U.reshape(B * H, W * C)
    out = _sc_sample(T2, scal, wq0a, wq1a, yoa, wy0a, wy1a, B, H, W, C)
    return out.reshape(B, C, oh, ow)


# speedup vs baseline: 1.0055x; 1.0055x over previous
import functools
import jax
import jax.numpy as jnp
from jax import lax
from jax.experimental import pallas as pl
from jax.experimental.pallas import tpu as pltpu
from jax.experimental.pallas import tpu_sc as plsc

NC, NS, L = 2, 16, 16
NW = NC * NS
NGRP = 16
NHALF = 2
NB = 4
PF = 2


def _sc_sample(T2, scal, wq0a, wq1a, yoa, wy0a, wy1a, B, H, W, C):
    P = H * W
    NI = H // NGRP
    JW = W // NHALF
    WW = JW + 2
    RWORDS = WW * C
    OWORDS = JW * C
    JPAD = wq0a.shape[-1]
    IPAD = yoa.shape[-1]

    mesh = plsc.VectorSubcoreMesh(core_axis_name="c", subcore_axis_name="s")

    @functools.partial(
        pl.kernel,
        mesh=mesh,
        out_type=jax.ShapeDtypeStruct((B * P * C,), jnp.float32),
        compiler_params=pltpu.CompilerParams(
            needs_layout_passes=False, use_tc_tiling_on_sc=False
        ),
        scratch_types=[
            pltpu.VMEM((16,), jnp.int32),
            pltpu.VMEM((JW * L,), jnp.float32),
            pltpu.VMEM((JW * L,), jnp.float32),
            pltpu.VMEM((IPAD,), jnp.int32),
            pltpu.VMEM((IPAD,), jnp.float32),
            pltpu.VMEM((IPAD,), jnp.float32),
            pltpu.VMEM((NB * RWORDS,), jnp.float32),
            pltpu.VMEM((2 * OWORDS,), jnp.float32),
            pltpu.SemaphoreType.DMA,
            pltpu.SemaphoreType.DMA,
        ],
    )
    def k(T2_hbm, scal_hbm, wq0_hbm, wq1_hbm, yo_hbm, wy0_hbm, wy1_hbm,
          out_hbm, scal_v, wq0ev, wq1ev, yov, wy0v, wy1v, RB, OB, rsem, osem):
        wid = lax.axis_index("s") * NC + lax.axis_index("c")
        g = wid // NHALF
        h = wid % NHALF
        i0 = g * NI
        j0 = h * JW

        def fire_row(b, sy, tx, r):
            src = T2_hbm.at[b * H + sy + r, pl.ds(tx * C, RWORDS)]
            pltpu.async_copy(src, RB.at[pl.ds((r % NB) * RWORDS, RWORDS)],
                             rsem)

        def wait_row(b):
            pltpu.make_async_copy(
                T2_hbm.at[b * H, pl.ds(0, RWORDS)],
                RB.at[pl.ds(0, RWORDS)], rsem
            ).wait()

        def fire_out(b, i_abs, p):
            dst = out_hbm.at[pl.ds((b * P + i_abs * W + j0) * C, OWORDS)]
            pltpu.async_copy(OB.at[pl.ds(p * OWORDS, OWORDS)], dst, osem)

        def drain_out():
            pltpu.make_async_copy(
                OB.at[pl.ds(0, OWORDS)], out_hbm.at[pl.ds(0, OWORDS)], osem
            ).wait()

        def batch(b, carry0):
            pltpu.sync_copy(scal_hbm.at[b, wid], scal_v)
            pltpu.sync_copy(wq0_hbm.at[b, h], wq0ev)
            pltpu.sync_copy(wq1_hbm.at[b, h], wq1ev)
            pltpu.sync_copy(yo_hbm.at[b, g], yov)
            pltpu.sync_copy(wy0_hbm.at[b, g], wy0v)
            pltpu.sync_copy(wy1_hbm.at[b, g], wy1v)
            sv = scal_v[pl.ds(0, L)]
            tx = sv[0]
            sy = sv[1]
            arel = sv[2]
            cl = sv[3]
            ch = sv[4]
            jlo8 = sv[5]
            ng = sv[6]
            jend = sv[7]
            lastv = yov[pl.ds(NI - 1, L)]
            cap = lastv[0] + 2

            def row(i_loc, carry):
                fired, waited = carry
                yv0 = yov[pl.ds(i_loc, L)]
                o0 = yv0[0]
                need = o0 + 2
                want = jnp.minimum(need + PF, cap)

                def fcond(s):
                    return s < want

                def fbody(s):
                    fire_row(b, sy, tx, s)
                    return s + 1

                fired = lax.while_loop(fcond, fbody, fired)

                def wcond(s):
                    return s < need

                def wbody(s):
                    wait_row(b)
                    return s + 1

                waited = lax.while_loop(wcond, wbody, waited)

                r0 = (o0 % NB) * RWORDS
                r1 = ((o0 + 1) % NB) * RWORDS
                wv0 = wy0v[pl.ds(i_loc, L)]
                wv1 = wy1v[pl.ds(i_loc, L)]
                gy0 = jnp.full((L,), wv0[0], dtype=jnp.float32)
                gy1 = jnp.full((L,), wv1[0], dtype=jnp.float32)
                p = i_loc % 2

                t_glob = b * NI + i_loc

                @pl.when(t_glob >= 2)
                def _():
                    drain_out()

                po = p * OWORDS

                def jone(jl, c2):
                    m = jnp.minimum(jnp.maximum(jl + arel, cl), ch) * C
                    t0 = wq0ev[pl.ds(jl * L, L)]
                    t1 = wq1ev[pl.ds(jl * L, L)]
                    wA = gy0 * t0
                    wB = gy1 * t0
                    wC = gy0 * t1
                    wD = gy1 * t1
                    ob = po + jl * C
                    for cc in range(C // L):
                        va = RB[pl.ds(r0 + m + cc * L, L)]
                        vb = RB[pl.ds(r1 + m + cc * L, L)]
                        vc = RB[pl.ds(r0 + m + C + cc * L, L)]
                        vd = RB[pl.ds(r1 + m + C + cc * L, L)]
                        acc = (wA * va + wB * vb) + (wC * vc + wD * vd)
                        OB[pl.ds(ob + cc * L, L)] = acc
                    return c2

                GRP = 8

                def jgroup(t, c2):
                    js = jlo8 + t * GRP
                    mb = (js + arel) * C
                    NCC = C // L
                    cur0 = [RB[pl.ds(r0 + mb + cc * L, L)]
                            for cc in range(NCC)]
                    cur1 = [RB[pl.ds(r1 + mb + cc * L, L)]
                            for cc in range(NCC)]
                    for kk in range(GRP):
                        nb = mb + (kk + 1) * C
                        nxt0 = [RB[pl.ds(r0 + nb + cc * L, L)]
                                for cc in range(NCC)]
                        nxt1 = [RB[pl.ds(r1 + nb + cc * L, L)]
                                for cc in range(NCC)]
                        jl = js + kk
                        t0 = wq0ev[pl.ds(jl * L, L)]
                        t1 = wq1ev[pl.ds(jl * L, L)]
                        wA = gy0 * t0
                        wB = gy1 * t0
                        wC = gy0 * t1
                        wD = gy1 * t1
                        ob = po + jl * C
                        for cc in range(NCC):
                            acc = (wA * cur0[cc] + wB * cur1[cc]) + \
                                (wC * nxt0[cc] + wD * nxt1[cc])
                            OB[pl.ds(ob + cc * L, L)] = acc
                        cur0 = nxt0
                        cur1 = nxt1
                    return c2

                lax.fori_loop(0, jlo8, jone, 0)
                lax.fori_loop(0, ng, jgroup, 0)
                lax.fori_loop(jend, JW, jone, 0)
                fire_out(b, i0 + i_loc, p)
                return (fired, waited)

            lax.fori_loop(0, NI, row, (jnp.int32(0), jnp.int32(0)))
            return carry0

        lax.fori_loop(0, B, batch, 0)
        drain_out()
        drain_out()

    return k(T2, scal, wq0a, wq1a, yoa, wy0a, wy1a)


def kernel(U, theta, out_size):
    B, C, H, W = U.shape
    oh, ow = H, W
    P = H * W
    NI = H // NGRP
    JW = W // NHALF
    WW = JW + 2
    NR = NI + 2
    zero = (jnp.asarray(out_size) - oh).astype(U.dtype)
    ox = jnp.linspace(-1.0, 1.0, ow)
    oy = jnp.linspace(-1.0, 1.0, oh)
    x = (theta[:, 0, 0][:, None] + ox[None, :]) + zero
    y = (theta[:, 1, 0][:, None] + oy[None, :]) + zero
    x = (x + 1.0) * (float(W) - 1.0) / 2.0
    y = (y + 1.0) * (float(H) - 1.0) / 2.0
    x0 = jnp.clip(jnp.floor(x).astype(jnp.int32), 0, W - 2)
    x1 = jnp.clip(jnp.ceil(x).astype(jnp.int32), 0, W - 1)
    y0 = jnp.clip(jnp.floor(y).astype(jnp.int32), 0, H - 2)
    y1 = jnp.clip(jnp.ceil(y).astype(jnp.int32), 0, H - 1)
    wx0 = x1.astype(x.dtype) - x
    wx1 = x - x0.astype(x.dtype)
    wy0 = y1.astype(y.dtype) - y
    wy1 = y - y0.astype(y.dtype)

    qoffx = jnp.floor(theta[:, 0, 0] * (float(W) - 1.0) / 2.0).astype(
        jnp.int32)
    qoffy = jnp.floor(theta[:, 1, 0] * (float(H) - 1.0) / 2.0).astype(
        jnp.int32)
    jj = jnp.arange(W, dtype=jnp.int32)
    ii = jnp.arange(H, dtype=jnp.int32)
    qx = jnp.clip(jj[None, :] + qoffx[:, None], 0, W - 2)
    qy = jnp.clip(ii[None, :] + qoffy[:, None], 0, H - 2)
    fz = jnp.float32(0.0)
    wq0 = jnp.where(x0 == qx, wx0, fz) + jnp.where(x1 == qx, wx1, fz)
    wq1 = jnp.where(x0 == qx + 1, wx0, fz) + jnp.where(x1 == qx + 1, wx1, fz)
    wyq0 = jnp.where(y0 == qy, wy0, fz) + jnp.where(y1 == qy, wy1, fz)
    wyq1 = jnp.where(y0 == qy + 1, wy0, fz) + jnp.where(y1 == qy + 1, wy1, fz)

    tx = jnp.minimum(qx[:, ::JW], W - WW)
    sy = jnp.minimum(qy[:, ::NI], H - NR)
    gg = jnp.arange(NW, dtype=jnp.int32) // NHALF
    hh = jnp.arange(NW, dtype=jnp.int32) % NHALF
    arel = hh[None, :] * JW + qoffx[:, None] - tx[:, hh]
    j0h = (hh * JW)[None, :]
    jlo_raw = jnp.clip(-qoffx[:, None] - j0h, 0, JW)
    jhi_raw = jnp.clip((W - 1) - qoffx[:, None] - j0h, 0, JW)
    jlo8 = jnp.minimum((jlo_raw + 7) // 8 * 8, JW)
    ngrp8 = jnp.maximum(jhi_raw - jlo8, 0) // 8
    jend = jlo8 + 8 * ngrp8
    scal = jnp.zeros((B, NW, 16), jnp.int32)
    scal = scal.at[:, :, 0].set(tx[:, hh])
    scal = scal.at[:, :, 1].set(sy[:, gg])
    scal = scal.at[:, :, 2].set(arel)
    scal = scal.at[:, :, 3].set(-tx[:, hh])
    scal = scal.at[:, :, 4].set((W - 2) - tx[:, hh])
    scal = scal.at[:, :, 5].set(jlo8)
    scal = scal.at[:, :, 6].set(ngrp8)
    scal = scal.at[:, :, 7].set(jend)

    wq0a = jnp.broadcast_to(
        wq0.reshape(B, NHALF, JW, 1), (B, NHALF, JW, 16)
    ).reshape(B, NHALF, JW * 16)
    wq1a = jnp.broadcast_to(
        wq1.reshape(B, NHALF, JW, 1), (B, NHALF, JW, 16)
    ).reshape(B, NHALF, JW * 16)

    IPAD = NI + 24
    yo = qy.reshape(B, NGRP, NI) - sy[:, :, None]
    yoa = jnp.zeros((B, NGRP, IPAD), jnp.int32).at[:, :, :NI].set(yo)
    wy0a = jnp.zeros((B, NGRP, IPAD), jnp.float32).at[:, :, :NI].set(
        wyq0.reshape(B, NGRP, NI))
    wy1a = jnp.zeros((B, NGRP, IPAD), jnp.float32).at[:, :, :NI].set(
        wyq1.reshape(B, NGRP, NI))

    T2 = U.reshape(B * H, W * C)
    out = _sc_sample(T2, scal, wq0a, wq1a, yoa, wy0a, wy1a, B, H, W, C)
    return out.reshape(B, C, oh, ow)
